# Initial kernel scaffold; baseline (speedup 1.0000x reference)
import functools

import jax
import jax.numpy as jnp
from jax import lax
from jax.experimental import pallas as pl
from jax.experimental.pallas import tpu as pltpu
from jax.experimental.pallas import tpu_sc as plsc

D = 64
NC = 2
NS = 16
NW = NC * NS
G = 128
K = 8
CHUNK = K * G


def _emb_body(b_per_w, table_hbm, idx_hbm, out_hbm, idx_v, rows_v, sem):
    wid = lax.axis_index("s") * NC + lax.axis_index("c")
    base = wid * b_per_w
    nblk = b_per_w // CHUNK

    def blk_body(blk, carry):
        blk_base = base + blk * CHUNK
        pltpu.sync_copy(idx_hbm.at[pl.ds(blk_base, CHUNK)], idx_v)
        copies = [
            pltpu.async_copy(
                table_hbm.at[idx_v.at[pl.ds(j * G, G)]],
                rows_v.at[pl.ds(j * G, G)],
                sem,
            )
            for j in range(K)
        ]
        for c in copies:
            c.wait()
        pltpu.sync_copy(rows_v, out_hbm.at[pl.ds(blk_base, CHUNK)])
        return carry

    lax.fori_loop(0, nblk, blk_body, 0)


def kernel(x, table):
    B, H = x.shape
    total = B * H
    b_per_w = total // NW
    idx_flat = x.reshape(total)

    mesh = plsc.VectorSubcoreMesh(core_axis_name="c", subcore_axis_name="s")
    emb = pl.kernel(
        functools.partial(_emb_body, b_per_w),
        out_type=jax.ShapeDtypeStruct((total, D), jnp.float32),
        mesh=mesh,
        scratch_types=[
            pltpu.VMEM((CHUNK,), jnp.int32),
            pltpu.VMEM((CHUNK, D), jnp.float32),
            pltpu.SemaphoreType.DMA,
        ],
    )
    out_flat = emb(table, idx_flat)
    return out_flat.reshape(B, H, D)


# speedup vs baseline: 1.8451x; 1.8451x over previous
import functools

import jax
import jax.numpy as jnp
from jax import lax
from jax.experimental import pallas as pl
from jax.experimental.pallas import tpu as pltpu
from jax.experimental.pallas import tpu_sc as plsc

D = 64
NC = 2
NS = 16
NW = NC * NS
G = 128
K = 8
CHUNK = K * G


def _emb_body(b_per_w, table_hbm, idx_hbm, out_hbm, idx_v, rows_v, sem):
    wid = lax.axis_index("s") * NC + lax.axis_index("c")
    base = wid * b_per_w
    nblk = b_per_w // CHUNK

    def blk_body(blk, carry):
        blk_base = base + blk * CHUNK
        pltpu.sync_copy(idx_hbm.at[pl.ds(blk_base, CHUNK)], idx_v)
        copies = [
            pltpu.async_copy(
                table_hbm.at[idx_v.at[pl.ds(j * G, G)]],
                rows_v.at[pl.ds(j * G, G)],
                sem,
            )
            for j in range(K)
        ]
        for c in copies:
            c.wait()
        pltpu.sync_copy(rows_v, out_hbm.at[pl.ds(blk_base, CHUNK)])
        return carry

    lax.fori_loop(0, nblk, blk_body, 0)


def kernel(x, table):
    B, H = x.shape
    total = B * H
    b_per_w = total // NW
    idx_flat = x.reshape(total)

    mesh = plsc.VectorSubcoreMesh(core_axis_name="c", subcore_axis_name="s")
    emb = pl.kernel(
        functools.partial(_emb_body, b_per_w),
        out_type=jax.ShapeDtypeStruct((total, D), jnp.float32),
        mesh=mesh,
        scratch_types=[
            pltpu.VMEM((CHUNK,), jnp.int32),
            pltpu.VMEM((CHUNK, D), jnp.float32),
            pltpu.SemaphoreType.DMA,
        ],
        compiler_params=pltpu.CompilerParams(use_tc_tiling_on_sc=False),
    )
    out_flat = emb(table, idx_flat)
    return out_flat.reshape(B, H, D)


# speedup vs baseline: 1.8768x; 1.0172x over previous
import functools

import jax
import jax.numpy as jnp
from jax import lax
from jax.experimental import pallas as pl
from jax.experimental.pallas import tpu as pltpu
from jax.experimental.pallas import tpu_sc as plsc

D = 64
NC = 2
NS = 16
NW = NC * NS
G = 128
K = 4
CHUNK = K * G


def _emb_body(b_per_w, table, idx_hbm, out,
              idx0, idx1, rows0, rows1, si0, si1, sg0, sg1, ss0, ss1):
    wid = lax.axis_index("s") * NC + lax.axis_index("c")
    base = wid * b_per_w
    nblk = b_per_w // CHUNK

    def idx_load(i, idxb, semb):
        pltpu.async_copy(idx_hbm.at[pl.ds(base + i * CHUNK, CHUNK)], idxb, semb)

    def wait_idx(idxb, semb):
        pltpu.make_async_copy(idx_hbm.at[pl.ds(0, CHUNK)], idxb, semb).wait()

    def fire(idxb, rowsb, semb):
        for j in range(K):
            pltpu.async_copy(
                table.at[idxb.at[pl.ds(j * G, G)]],
                rowsb.at[pl.ds(j * G, G)],
                semb,
            )

    def wait_g(rowsb, semb):
        pltpu.make_async_copy(table.at[pl.ds(0, CHUNK)], rowsb, semb).wait()

    def store(i, rowsb, semb):
        pltpu.async_copy(rowsb, out.at[pl.ds(base + i * CHUNK, CHUNK)], semb)

    def wait_store(rowsb, semb):
        pltpu.make_async_copy(rowsb, out.at[pl.ds(0, CHUNK)], semb).wait()

    idx_load(0, idx0, si0)
    idx_load(1, idx1, si1)
    wait_idx(idx0, si0)
    fire(idx0, rows0, sg0)
    wait_idx(idx1, si1)
    fire(idx1, rows1, sg1)
    wait_g(rows0, sg0)
    store(0, rows0, ss0)
    idx_load(2, idx0, si0)

    def pair(p, carry):
        i0 = 2 * p
        wait_idx(idx0, si0)
        wait_store(rows0, ss0)
        fire(idx0, rows0, sg0)
        wait_g(rows1, sg1)
        store(i0 - 1, rows1, ss1)
        idx_load(i0 + 1, idx1, si1)

        wait_idx(idx1, si1)
        wait_store(rows1, ss1)
        fire(idx1, rows1, sg1)
        wait_g(rows0, sg0)
        store(i0, rows0, ss0)
        idx_load(i0 + 2, idx0, si0)
        return carry

    lax.fori_loop(1, nblk // 2 - 1, pair, 0)

    i0 = nblk - 2
    wait_idx(idx0, si0)
    wait_store(rows0, ss0)
    fire(idx0, rows0, sg0)
    wait_g(rows1, sg1)
    store(i0 - 1, rows1, ss1)
    idx_load(i0 + 1, idx1, si1)

    wait_idx(idx1, si1)
    wait_store(rows1, ss1)
    fire(idx1, rows1, sg1)
    wait_g(rows0, sg0)
    store(i0, rows0, ss0)

    wait_g(rows1, sg1)
    store(i0 + 1, rows1, ss1)
    wait_store(rows0, ss0)
    wait_store(rows1, ss1)


def kernel(x, table):
    B, H = x.shape
    total = B * H
    b_per_w = total // NW
    assert total % NW == 0
    nblk = b_per_w // CHUNK
    assert b_per_w % CHUNK == 0 and nblk % 2 == 0 and nblk >= 4
    idx_flat = x.reshape(total)

    mesh = plsc.VectorSubcoreMesh(core_axis_name="c", subcore_axis_name="s")
    emb = pl.kernel(
        functools.partial(_emb_body, b_per_w),
        out_type=jax.ShapeDtypeStruct((total, D), jnp.float32),
        mesh=mesh,
        scratch_types=[
            pltpu.VMEM((CHUNK,), jnp.int32),
            pltpu.VMEM((CHUNK,), jnp.int32),
            pltpu.VMEM((CHUNK, D), jnp.float32),
            pltpu.VMEM((CHUNK, D), jnp.float32),
            pltpu.SemaphoreType.DMA,
            pltpu.SemaphoreType.DMA,
            pltpu.SemaphoreType.DMA,
            pltpu.SemaphoreType.DMA,
            pltpu.SemaphoreType.DMA,
            pltpu.SemaphoreType.DMA,
        ],
        compiler_params=pltpu.CompilerParams(use_tc_tiling_on_sc=False),
    )
    out_flat = emb(table, idx_flat)
    return out_flat.reshape(B, H, D)
